# initial kernel scaffold (unmeasured)
import jax
import jax.numpy as jnp
from jax import lax
from jax.experimental import pallas as pl
from jax.experimental.pallas import tpu as pltpu

B, H, D, BS = 8, 8, 128, 16
NP_LOCAL = 512
G = 8
SCALE = D ** -0.5
NEG = -1e30


def kernel(Q, K, V, bt, lens):
    q = Q.reshape(B, H, D)
    lens2 = lens.reshape(B, 1)

    def body(q_ref, k_ref, v_ref, bt_ref, lens_ref, out_ref,
             kbuf, vbuf, ksems, vsems, comm_send, comm_recv,
             send_sem, recv_sem):
        my_x = lax.axis_index("x")
        my_y = lax.axis_index("y")
        peer = (1 - my_x, my_y)

        barrier = pltpu.get_barrier_semaphore()
        pl.semaphore_signal(barrier, inc=1, device_id=peer,
                            device_id_type=pl.DeviceIdType.MESH)
        pl.semaphore_wait(barrier, 1)

        kbuf[...] = jnp.zeros_like(kbuf)
        vbuf[...] = jnp.zeros_like(vbuf)

        page_base = my_x * NP_LOCAL

        for i in range(B):
            len_i = lens_ref[i, 0]
            n_steps = (len_i + G - 1) // G
            q_i = q_ref[i]

            def step_fn(t, carry, i=i, len_i=len_i):
                m, l, acc = carry
                base = t * G
                owned_flags = []
                for g in range(G):
                    j = base + g
                    lidx = bt_ref[i, j] - page_base
                    owned = (j < len_i) & (lidx >= 0) & (lidx < NP_LOCAL)
                    owned_flags.append(owned)

                    @pl.when(owned)
                    def _(lidx=lidx, g=g):
                        pltpu.make_async_copy(
                            k_ref.at[lidx], kbuf.at[g], ksems.at[g]).start()
                        pltpu.make_async_copy(
                            v_ref.at[lidx], vbuf.at[g], vsems.at[g]).start()

                for g in range(G):
                    @pl.when(owned_flags[g])
                    def _(g=g):
                        pltpu.make_async_copy(
                            k_ref.at[0], kbuf.at[g], ksems.at[g]).wait()
                        pltpu.make_async_copy(
                            v_ref.at[0], vbuf.at[g], vsems.at[g]).wait()

                k_all = kbuf[...].reshape(G * BS, H, D)
                v_all = vbuf[...].reshape(G * BS, H, D)
                s = lax.dot_general(
                    q_i, k_all,
                    (((1,), (2,)), ((0,), (1,))),
                    preferred_element_type=jnp.float32,
                ) * SCALE

                gidx = lax.broadcasted_iota(jnp.int32, (H, G * BS), 1) // BS
                mask = jnp.zeros((H, G * BS), dtype=jnp.bool_)
                for g in range(G):
                    mask = mask | ((gidx == g) & owned_flags[g])

                s = jnp.where(mask, s, NEG)
                m_new = jnp.maximum(m, jnp.max(s, axis=1, keepdims=True))
                alpha = jnp.exp(m - m_new)
                p = jnp.where(mask, jnp.exp(s - m_new), 0.0)
                l_new = l * alpha + jnp.sum(p, axis=1, keepdims=True)
                pv = lax.dot_general(
                    p, v_all,
                    (((1,), (0,)), ((0,), (1,))),
                    preferred_element_type=jnp.float32,
                )
                acc_new = acc * alpha + pv
                return m_new, l_new, acc_new

            m0 = jnp.full((H, 1), NEG, dtype=jnp.float32)
            l0 = jnp.zeros((H, 1), dtype=jnp.float32)
            a0 = jnp.zeros((H, D), dtype=jnp.float32)
            m, l, acc = lax.fori_loop(0, n_steps, step_fn, (m0, l0, a0))

            comm_send[0, i] = acc
            comm_send[1, i] = jnp.broadcast_to(m, (H, D))
            comm_send[2, i] = jnp.broadcast_to(l, (H, D))

        rdma = pltpu.make_async_remote_copy(
            src_ref=comm_send, dst_ref=comm_recv,
            send_sem=send_sem, recv_sem=recv_sem,
            device_id=peer, device_id_type=pl.DeviceIdType.MESH,
        )
        rdma.start()
        rdma.wait()

        acc_a, m_a, l_a = comm_send[0], comm_send[1], comm_send[2]
        acc_b, m_b, l_b = comm_recv[0], comm_recv[1], comm_recv[2]
        m_star = jnp.maximum(m_a, m_b)
        aa = jnp.exp(m_a - m_star)
        ab = jnp.exp(m_b - m_star)
        l_star = l_a * aa + l_b * ab
        out_ref[...] = (acc_a * aa + acc_b * ab) / l_star

    out = pl.pallas_call(
        body,
        out_shape=jax.ShapeDtypeStruct((B, H, D), jnp.float32),
        in_specs=[
            pl.BlockSpec(memory_space=pltpu.VMEM),
            pl.BlockSpec(memory_space=pltpu.ANY),
            pl.BlockSpec(memory_space=pltpu.ANY),
            pl.BlockSpec(memory_space=pltpu.SMEM),
            pl.BlockSpec(memory_space=pltpu.SMEM),
        ],
        out_specs=pl.BlockSpec(memory_space=pltpu.VMEM),
        scratch_shapes=[
            pltpu.VMEM((G, BS, H, D), jnp.float32),
            pltpu.VMEM((G, BS, H, D), jnp.float32),
            pltpu.SemaphoreType.DMA((G,)),
            pltpu.SemaphoreType.DMA((G,)),
            pltpu.VMEM((3, B, H, D), jnp.float32),
            pltpu.VMEM((3, B, H, D), jnp.float32),
            pltpu.SemaphoreType.DMA,
            pltpu.SemaphoreType.DMA,
        ],
        compiler_params=pltpu.CompilerParams(collective_id=0),
    )(q, K, V, bt, lens2)
    return out.reshape(B, 1, H, D)


# baseline (device time: 1157831 ns/iter reference)
import jax
import jax.numpy as jnp
from jax import lax
from jax.experimental import pallas as pl
from jax.experimental.pallas import tpu as pltpu

B, H, D, BS = 8, 8, 128, 16
NP_LOCAL = 512
G = 8
SCALE = D ** -0.5
NEG = -1e30


def kernel(Q, K, V, bt, lens):
    q = Q.reshape(B, H, D)
    lens2 = lens.reshape(B, 1)

    def body(q_ref, k_ref, v_ref, bt_ref, lens_ref, out_ref,
             kbuf, vbuf, ksems, vsems, comm_send, comm_recv,
             send_sem, recv_sem):
        my_x = lax.axis_index("x")
        my_y = lax.axis_index("y")
        peer = (1 - my_x, my_y)

        barrier = pltpu.get_barrier_semaphore()
        pl.semaphore_signal(barrier, inc=1, device_id=peer,
                            device_id_type=pl.DeviceIdType.MESH)
        pl.semaphore_wait(barrier, 1)

        kbuf[...] = jnp.zeros_like(kbuf)
        vbuf[...] = jnp.zeros_like(vbuf)

        page_base = my_x * NP_LOCAL

        for i in range(B):
            len_i = lens_ref[i, 0]
            n_steps = (len_i + G - 1) // G
            q_i = q_ref[i]

            def step_fn(t, carry, i=i, len_i=len_i):
                m, l, acc = carry
                base = t * G
                owned_flags = []
                for g in range(G):
                    j = base + g
                    lidx = bt_ref[i, j] - page_base
                    owned = (j < len_i) & (lidx >= 0) & (lidx < NP_LOCAL)
                    owned_flags.append(owned)

                    @pl.when(owned)
                    def _(lidx=lidx, g=g):
                        pltpu.make_async_copy(
                            k_ref.at[lidx], kbuf.at[g], ksems.at[g]).start()
                        pltpu.make_async_copy(
                            v_ref.at[lidx], vbuf.at[g], vsems.at[g]).start()

                for g in range(G):
                    @pl.when(owned_flags[g])
                    def _(g=g):
                        pltpu.make_async_copy(
                            k_ref.at[0], kbuf.at[g], ksems.at[g]).wait()
                        pltpu.make_async_copy(
                            v_ref.at[0], vbuf.at[g], vsems.at[g]).wait()

                k_all = kbuf[...].reshape(G * BS, H, D)
                v_all = vbuf[...].reshape(G * BS, H, D)
                s = lax.dot_general(
                    q_i, k_all,
                    (((1,), (2,)), ((0,), (1,))),
                    preferred_element_type=jnp.float32,
                ) * SCALE

                gidx = lax.broadcasted_iota(jnp.int32, (H, G * BS), 1) // BS
                mask = jnp.zeros((H, G * BS), dtype=jnp.bool_)
                for g in range(G):
                    mask = mask | ((gidx == g) & owned_flags[g])

                s = jnp.where(mask, s, NEG)
                m_new = jnp.maximum(m, jnp.max(s, axis=1, keepdims=True))
                alpha = jnp.exp(m - m_new)
                p = jnp.where(mask, jnp.exp(s - m_new), 0.0)
                l_new = l * alpha + jnp.sum(p, axis=1, keepdims=True)
                pv = lax.dot_general(
                    p, v_all,
                    (((1,), (0,)), ((0,), (1,))),
                    preferred_element_type=jnp.float32,
                )
                acc_new = acc * alpha + pv
                return m_new, l_new, acc_new

            m0 = jnp.full((H, 1), NEG, dtype=jnp.float32)
            l0 = jnp.zeros((H, 1), dtype=jnp.float32)
            a0 = jnp.zeros((H, D), dtype=jnp.float32)
            m, l, acc = lax.fori_loop(0, n_steps, step_fn, (m0, l0, a0))

            comm_send[0, i] = acc
            comm_send[1, i] = jnp.broadcast_to(m, (H, D))
            comm_send[2, i] = jnp.broadcast_to(l, (H, D))

        rdma = pltpu.make_async_remote_copy(
            src_ref=comm_send, dst_ref=comm_recv,
            send_sem=send_sem, recv_sem=recv_sem,
            device_id=peer, device_id_type=pl.DeviceIdType.MESH,
        )
        rdma.start()
        rdma.wait()

        acc_a, m_a, l_a = comm_send[0], comm_send[1], comm_send[2]
        acc_b, m_b, l_b = comm_recv[0], comm_recv[1], comm_recv[2]
        m_star = jnp.maximum(m_a, m_b)
        aa = jnp.exp(m_a - m_star)
        ab = jnp.exp(m_b - m_star)
        l_star = l_a * aa + l_b * ab
        out_ref[...] = (acc_a * aa + acc_b * ab) / l_star

    out = pl.pallas_call(
        body,
        out_shape=jax.ShapeDtypeStruct((B, H, D), jnp.float32),
        in_specs=[
            pl.BlockSpec(memory_space=pltpu.VMEM),
            pl.BlockSpec(memory_space=pl.ANY),
            pl.BlockSpec(memory_space=pl.ANY),
            pl.BlockSpec(memory_space=pltpu.SMEM),
            pl.BlockSpec(memory_space=pltpu.SMEM),
        ],
        out_specs=pl.BlockSpec(memory_space=pltpu.VMEM),
        scratch_shapes=[
            pltpu.VMEM((G, BS, H, D), jnp.float32),
            pltpu.VMEM((G, BS, H, D), jnp.float32),
            pltpu.SemaphoreType.DMA((G,)),
            pltpu.SemaphoreType.DMA((G,)),
            pltpu.VMEM((3, B, H, D), jnp.float32),
            pltpu.VMEM((3, B, H, D), jnp.float32),
            pltpu.SemaphoreType.DMA,
            pltpu.SemaphoreType.DMA,
        ],
        compiler_params=pltpu.CompilerParams(collective_id=0),
    )(q, K, V, bt, lens2)
    return out.reshape(B, 1, H, D)


# device time: 402307 ns/iter; 2.8780x vs baseline; 2.8780x over previous
import jax
import jax.numpy as jnp
from jax import lax
from jax.experimental import pallas as pl
from jax.experimental.pallas import tpu as pltpu

B, H, D, BS = 8, 8, 128, 16
NP_LOCAL = 512
MAXLEN = 512
NB = B // 2
G = 16
SCALE = D ** -0.5
NEG = -1e30


def kernel(Q, K, V, bt, lens):
    my_x = lax.axis_index("x")
    my_y = lax.axis_index("y")
    q = Q.reshape(B, H, D)

    jidx = jnp.arange(MAXLEN, dtype=jnp.int32)[None, :]
    lidx = bt - my_x * NP_LOCAL
    owned = (jidx < lens[:, None]) & (lidx >= 0) & (lidx < NP_LOCAL)
    order = jnp.argsort(~owned, axis=1, stable=True)
    bt_local = jnp.take_along_axis(
        jnp.clip(lidx, 0, NP_LOCAL - 1), order, axis=1
    ).astype(jnp.int32)
    counts = jnp.sum(owned, axis=1).astype(jnp.int32)

    base = my_y * NB
    q_my = lax.dynamic_slice(q, (base, 0, 0), (NB, H, D))
    bt_my = lax.dynamic_slice(bt_local, (base, 0), (NB, MAXLEN))
    counts_my = lax.dynamic_slice(counts, (base,), (NB,)).reshape(NB, 1)

    def body(q_ref, k_ref, v_ref, bt_ref, counts_ref, out_ref,
             kbuf, vbuf, ksems, vsems, comm_send, comm_recv,
             send_sem, recv_sem, send_sem_y, recv_sem_y):
        my_x = lax.axis_index("x")
        my_y = lax.axis_index("y")
        x_peer = (1 - my_x, my_y)
        y_peer = (my_x, 1 - my_y)

        barrier = pltpu.get_barrier_semaphore()
        for nbr in (x_peer, y_peer):
            pl.semaphore_signal(barrier, inc=1, device_id=nbr,
                                device_id_type=pl.DeviceIdType.MESH)
        pl.semaphore_wait(barrier, 2)

        def start_group(i, t, slot):
            for g in range(G):
                pidx = bt_ref[i, t * G + g]
                pltpu.make_async_copy(
                    k_ref.at[pidx], kbuf.at[slot, g], ksems.at[slot]).start()
                pltpu.make_async_copy(
                    v_ref.at[pidx], vbuf.at[slot, g], vsems.at[slot]).start()

        for i in range(NB):
            count = counts_ref[i, 0]
            n_steps = (count + G - 1) // G
            q_i = q_ref[i]

            @pl.when(n_steps > 0)
            def _(i=i):
                start_group(i, 0, 0)

            def step_fn(t, carry, i=i, count=count, n_steps=n_steps):
                m, l, acc = carry
                slot = lax.rem(t, 2)

                @pl.when(t + 1 < n_steps)
                def _():
                    start_group(i, t + 1, 1 - slot)

                for _ in range(G):
                    pltpu.make_async_copy(
                        k_ref.at[0], kbuf.at[slot, 0], ksems.at[slot]).wait()
                    pltpu.make_async_copy(
                        v_ref.at[0], vbuf.at[slot, 0], vsems.at[slot]).wait()

                k_all = kbuf[slot].reshape(G * BS, H, D)
                v_all = vbuf[slot].reshape(G * BS, H, D)
                s = lax.dot_general(
                    q_i, k_all,
                    (((1,), (2,)), ((0,), (1,))),
                    preferred_element_type=jnp.float32,
                ) * SCALE

                gidx = lax.broadcasted_iota(jnp.int32, (H, G * BS), 1) // BS
                mask = (t * G + gidx) < count

                s = jnp.where(mask, s, NEG)
                m_new = jnp.maximum(m, jnp.max(s, axis=1, keepdims=True))
                alpha = jnp.exp(m - m_new)
                p = jnp.where(mask, jnp.exp(s - m_new), 0.0)
                l_new = l * alpha + jnp.sum(p, axis=1, keepdims=True)
                pv = lax.dot_general(
                    p, v_all,
                    (((1,), (0,)), ((0,), (1,))),
                    preferred_element_type=jnp.float32,
                )
                acc_new = acc * alpha + pv
                return m_new, l_new, acc_new

            m0 = jnp.full((H, 1), NEG, dtype=jnp.float32)
            l0 = jnp.zeros((H, 1), dtype=jnp.float32)
            a0 = jnp.zeros((H, D), dtype=jnp.float32)
            m, l, acc = lax.fori_loop(0, n_steps, step_fn, (m0, l0, a0))

            comm_send[0, i] = acc
            comm_send[1, i] = jnp.broadcast_to(m, (H, D))
            comm_send[2, i] = jnp.broadcast_to(l, (H, D))

        rdma_x = pltpu.make_async_remote_copy(
            src_ref=comm_send, dst_ref=comm_recv,
            send_sem=send_sem, recv_sem=recv_sem,
            device_id=x_peer, device_id_type=pl.DeviceIdType.MESH,
        )
        rdma_x.start()
        rdma_x.wait()

        acc_a, m_a, l_a = comm_send[0], comm_send[1], comm_send[2]
        acc_b, m_b, l_b = comm_recv[0], comm_recv[1], comm_recv[2]
        m_star = jnp.maximum(m_a, m_b)
        aa = jnp.exp(m_a - m_star)
        ab = jnp.exp(m_b - m_star)
        l_star = l_a * aa + l_b * ab
        row0 = my_y * NB
        out_ref[pl.ds(row0, NB)] = (acc_a * aa + acc_b * ab) / l_star

        rdma_y = pltpu.make_async_remote_copy(
            src_ref=out_ref.at[pl.ds(row0, NB)],
            dst_ref=out_ref.at[pl.ds(row0, NB)],
            send_sem=send_sem_y, recv_sem=recv_sem_y,
            device_id=y_peer, device_id_type=pl.DeviceIdType.MESH,
        )
        rdma_y.start()
        rdma_y.wait()

    out = pl.pallas_call(
        body,
        out_shape=jax.ShapeDtypeStruct((B, H, D), jnp.float32),
        in_specs=[
            pl.BlockSpec(memory_space=pltpu.VMEM),
            pl.BlockSpec(memory_space=pl.ANY),
            pl.BlockSpec(memory_space=pl.ANY),
            pl.BlockSpec(memory_space=pltpu.SMEM),
            pl.BlockSpec(memory_space=pltpu.SMEM),
        ],
        out_specs=pl.BlockSpec(memory_space=pltpu.VMEM),
        scratch_shapes=[
            pltpu.VMEM((2, G, BS, H, D), jnp.float32),
            pltpu.VMEM((2, G, BS, H, D), jnp.float32),
            pltpu.SemaphoreType.DMA((2,)),
            pltpu.SemaphoreType.DMA((2,)),
            pltpu.VMEM((3, NB, H, D), jnp.float32),
            pltpu.VMEM((3, NB, H, D), jnp.float32),
            pltpu.SemaphoreType.DMA,
            pltpu.SemaphoreType.DMA,
            pltpu.SemaphoreType.DMA,
            pltpu.SemaphoreType.DMA,
        ],
        compiler_params=pltpu.CompilerParams(collective_id=0),
    )(q_my, K, V, bt_my, counts_my)
    return out.reshape(B, 1, H, D)


# device time: 122414 ns/iter; 9.4583x vs baseline; 3.2864x over previous
import jax
import jax.numpy as jnp
from jax import lax
from jax.experimental import pallas as pl
from jax.experimental.pallas import tpu as pltpu

B, H, D, BS = 8, 8, 128, 16
HD = H * D
NP_LOCAL = 512
MAXLEN = 512
NB = B // 2
G = 32
K_STEP = G * BS
SCALE = D ** -0.5
NEG = -1e30


def kernel(Q, K, V, bt, lens):
    my_x = lax.axis_index("x")
    my_y = lax.axis_index("y")
    q = Q.reshape(B, H, D)
    k_flat = K.reshape(NP_LOCAL, BS, HD)
    v_flat = V.reshape(NP_LOCAL, BS, HD)

    jidx = jnp.arange(MAXLEN, dtype=jnp.int32)[None, :]
    lidx = bt - my_x * NP_LOCAL
    owned = (jidx < lens[:, None]) & (lidx >= 0) & (lidx < NP_LOCAL)
    bt_local = jnp.sort(jnp.where(owned, lidx, NP_LOCAL - 1), axis=1)
    counts = jnp.sum(owned, axis=1).astype(jnp.int32)

    row0 = my_y * NB
    q_my = lax.dynamic_slice(q, (row0, 0, 0), (NB, H, D))
    bt_my = lax.dynamic_slice(bt_local, (row0, 0), (NB, MAXLEN))
    counts_my = lax.dynamic_slice(counts, (row0,), (NB,)).reshape(NB, 1)

    def body(q_ref, k_ref, v_ref, bt_ref, counts_ref, out_ref,
             kbuf, vbuf, ksems, vsems, comm_send, comm_recv,
             send_sem, recv_sem, send_sem_y, recv_sem_y):
        my_x = lax.axis_index("x")
        my_y = lax.axis_index("y")
        x_peer = (1 - my_x, my_y)
        y_peer = (my_x, 1 - my_y)

        barrier = pltpu.get_barrier_semaphore()
        for nbr in (x_peer, y_peer):
            pl.semaphore_signal(barrier, inc=1, device_id=nbr,
                                device_id_type=pl.DeviceIdType.MESH)
        pl.semaphore_wait(barrier, 2)

        row_i = lax.broadcasted_iota(jnp.int32, (H, HD), 0)
        col_h = lax.broadcasted_iota(jnp.int32, (H, HD), 1) // D
        diag = row_i == col_h
        kiota = lax.broadcasted_iota(jnp.int32, (H, K_STEP), 1) // BS

        def start_group(i, t, slot):
            for g in range(G):
                pidx = bt_ref[i, t * G + g]
                pltpu.make_async_copy(
                    k_ref.at[pidx], kbuf.at[slot, g], ksems.at[slot]).start()
                pltpu.make_async_copy(
                    v_ref.at[pidx], vbuf.at[slot, g], vsems.at[slot]).start()

        for i in range(NB):
            count = counts_ref[i, 0]
            n_steps = (count + G - 1) // G

            q_tiled = jnp.concatenate([q_ref[i]] * H, axis=1)
            q_bd = jnp.where(diag, q_tiled, 0.0)

            @pl.when(n_steps > 0)
            def _(i=i):
                start_group(i, 0, 0)

            def step_fn(t, carry, i=i, count=count, n_steps=n_steps,
                        q_bd=q_bd):
                m, l, acc = carry
                slot = lax.rem(t, 2)

                @pl.when(t + 1 < n_steps)
                def _():
                    start_group(i, t + 1, 1 - slot)

                for _ in range(G):
                    pltpu.make_async_copy(
                        k_ref.at[0], kbuf.at[slot, 0], ksems.at[slot]).wait()
                    pltpu.make_async_copy(
                        v_ref.at[0], vbuf.at[slot, 0], vsems.at[slot]).wait()

                k_all = kbuf[slot].reshape(K_STEP, HD)
                v_all = vbuf[slot].reshape(K_STEP, HD)
                s = lax.dot_general(
                    q_bd, k_all,
                    (((1,), (1,)), ((), ())),
                    preferred_element_type=jnp.float32,
                ) * SCALE

                mask = (t * G + kiota) < count
                s = jnp.where(mask, s, NEG)
                m_new = jnp.maximum(m, jnp.max(s, axis=1, keepdims=True))
                alpha = jnp.exp(m - m_new)
                p = jnp.where(mask, jnp.exp(s - m_new), 0.0)
                l_new = l * alpha + jnp.sum(p, axis=1, keepdims=True)
                r = lax.dot_general(
                    p, v_all,
                    (((1,), (0,)), ((), ())),
                    preferred_element_type=jnp.float32,
                )
                rm = jnp.where(diag, r, 0.0)
                pv = rm[:, 0:D]
                for hh in range(1, H):
                    pv = pv + rm[:, hh * D:(hh + 1) * D]
                acc_new = acc * alpha + pv
                return m_new, l_new, acc_new

            m0 = jnp.full((H, 1), NEG, dtype=jnp.float32)
            l0 = jnp.zeros((H, 1), dtype=jnp.float32)
            a0 = jnp.zeros((H, D), dtype=jnp.float32)
            m, l, acc = lax.fori_loop(0, n_steps, step_fn, (m0, l0, a0))

            comm_send[0, i] = acc
            comm_send[1, i] = jnp.broadcast_to(m, (H, D))
            comm_send[2, i] = jnp.broadcast_to(l, (H, D))

        rdma_x = pltpu.make_async_remote_copy(
            src_ref=comm_send, dst_ref=comm_recv,
            send_sem=send_sem, recv_sem=recv_sem,
            device_id=x_peer, device_id_type=pl.DeviceIdType.MESH,
        )
        rdma_x.start()
        rdma_x.wait()

        acc_a, m_a, l_a = comm_send[0], comm_send[1], comm_send[2]
        acc_b, m_b, l_b = comm_recv[0], comm_recv[1], comm_recv[2]
        m_star = jnp.maximum(m_a, m_b)
        aa = jnp.exp(m_a - m_star)
        ab = jnp.exp(m_b - m_star)
        l_star = l_a * aa + l_b * ab
        base = my_y * NB
        out_ref[pl.ds(base, NB)] = (acc_a * aa + acc_b * ab) / l_star

        rdma_y = pltpu.make_async_remote_copy(
            src_ref=out_ref.at[pl.ds(base, NB)],
            dst_ref=out_ref.at[pl.ds(base, NB)],
            send_sem=send_sem_y, recv_sem=recv_sem_y,
            device_id=y_peer, device_id_type=pl.DeviceIdType.MESH,
        )
        rdma_y.start()
        rdma_y.wait()

    out = pl.pallas_call(
        body,
        out_shape=jax.ShapeDtypeStruct((B, H, D), jnp.float32),
        in_specs=[
            pl.BlockSpec(memory_space=pltpu.VMEM),
            pl.BlockSpec(memory_space=pl.ANY),
            pl.BlockSpec(memory_space=pl.ANY),
            pl.BlockSpec(memory_space=pltpu.SMEM),
            pl.BlockSpec(memory_space=pltpu.SMEM),
        ],
        out_specs=pl.BlockSpec(memory_space=pltpu.VMEM),
        scratch_shapes=[
            pltpu.VMEM((2, G, BS, HD), jnp.float32),
            pltpu.VMEM((2, G, BS, HD), jnp.float32),
            pltpu.SemaphoreType.DMA((2,)),
            pltpu.SemaphoreType.DMA((2,)),
            pltpu.VMEM((3, NB, H, D), jnp.float32),
            pltpu.VMEM((3, NB, H, D), jnp.float32),
            pltpu.SemaphoreType.DMA,
            pltpu.SemaphoreType.DMA,
            pltpu.SemaphoreType.DMA,
            pltpu.SemaphoreType.DMA,
        ],
        compiler_params=pltpu.CompilerParams(collective_id=0),
    )(q_my, k_flat, v_flat, bt_my, counts_my)
    return out.reshape(B, 1, H, D)


# device time: 54557 ns/iter; 21.2224x vs baseline; 2.2438x over previous
import jax
import jax.numpy as jnp
from jax import lax
from jax.experimental import pallas as pl
from jax.experimental.pallas import tpu as pltpu

B, H, D, BS = 8, 8, 128, 16
HD = H * D
NP_LOCAL = 512
MAXLEN = 512
NB = B // 2
G = 32
K_STEP = G * BS
SCALE = D ** -0.5
NEG = -1e30


def kernel(Q, K, V, bt, lens):
    my_x = lax.axis_index("x")
    my_y = lax.axis_index("y")
    q = Q.reshape(B, H, D)

    jidx = jnp.arange(MAXLEN, dtype=jnp.int32)[None, :]
    lidx = bt - my_x * NP_LOCAL
    owned = (jidx < lens[:, None]) & (lidx >= 0) & (lidx < NP_LOCAL)
    bt_local = jnp.sort(jnp.where(owned, lidx, NP_LOCAL - 1), axis=1)
    counts = jnp.sum(owned, axis=1).astype(jnp.int32)

    row0 = my_y * NB
    q_my = lax.dynamic_slice(q, (row0, 0, 0), (NB, H, D))
    bt_my = lax.dynamic_slice(bt_local, (row0, 0), (NB, MAXLEN))
    counts_my = lax.dynamic_slice(counts, (row0,), (NB,)).reshape(NB, 1)

    def body(q_ref, k_ref, v_ref, bt_ref, counts_ref, out_ref,
             kbuf, vbuf, ksems, vsems, comm_send, comm_recv,
             send_sem, recv_sem, send_sem_y, recv_sem_y):
        my_x = lax.axis_index("x")
        my_y = lax.axis_index("y")
        x_peer = (1 - my_x, my_y)
        y_peer = (my_x, 1 - my_y)

        barrier = pltpu.get_barrier_semaphore()
        for nbr in (x_peer, y_peer):
            pl.semaphore_signal(barrier, inc=1, device_id=nbr,
                                device_id_type=pl.DeviceIdType.MESH)
        pl.semaphore_wait(barrier, 2)

        kf_ref = k_ref.reshape(NP_LOCAL, BS, HD)
        vf_ref = v_ref.reshape(NP_LOCAL, BS, HD)

        row_i = lax.broadcasted_iota(jnp.int32, (H, HD), 0)
        col_h = lax.broadcasted_iota(jnp.int32, (H, HD), 1) // D
        diag = row_i == col_h
        kiota = lax.broadcasted_iota(jnp.int32, (H, K_STEP), 1) // BS

        def start_group(i, t, slot):
            for g in range(G):
                pidx = bt_ref[i, t * G + g]
                pltpu.make_async_copy(
                    kf_ref.at[pidx], kbuf.at[slot, g], ksems.at[slot]).start()
                pltpu.make_async_copy(
                    vf_ref.at[pidx], vbuf.at[slot, g], vsems.at[slot]).start()

        for i in range(NB):
            count = counts_ref[i, 0]
            n_steps = (count + G - 1) // G

            q_tiled = jnp.concatenate([q_ref[i]] * H, axis=1)
            q_bd = jnp.where(diag, q_tiled, 0.0)

            @pl.when(n_steps > 0)
            def _(i=i):
                start_group(i, 0, 0)

            def step_fn(t, carry, i=i, count=count, n_steps=n_steps,
                        q_bd=q_bd):
                m, l, acc = carry
                slot = lax.rem(t, 2)

                @pl.when(t + 1 < n_steps)
                def _():
                    start_group(i, t + 1, 1 - slot)

                for _ in range(G):
                    pltpu.make_async_copy(
                        kf_ref.at[0], kbuf.at[slot, 0], ksems.at[slot]).wait()
                    pltpu.make_async_copy(
                        vf_ref.at[0], vbuf.at[slot, 0], vsems.at[slot]).wait()

                k_all = kbuf[slot].reshape(K_STEP, HD)
                v_all = vbuf[slot].reshape(K_STEP, HD)
                s = lax.dot_general(
                    q_bd, k_all,
                    (((1,), (1,)), ((), ())),
                    preferred_element_type=jnp.float32,
                ) * SCALE

                mask = (t * G + kiota) < count
                s = jnp.where(mask, s, NEG)
                m_new = jnp.maximum(m, jnp.max(s, axis=1, keepdims=True))
                alpha = jnp.exp(m - m_new)
                p = jnp.where(mask, jnp.exp(s - m_new), 0.0)
                l_new = l * alpha + jnp.sum(p, axis=1, keepdims=True)
                r = lax.dot_general(
                    p, v_all,
                    (((1,), (0,)), ((), ())),
                    preferred_element_type=jnp.float32,
                )
                rm = jnp.where(diag, r, 0.0)
                pv = rm[:, 0:D]
                for hh in range(1, H):
                    pv = pv + rm[:, hh * D:(hh + 1) * D]
                acc_new = acc * alpha + pv
                return m_new, l_new, acc_new

            m0 = jnp.full((H, 1), NEG, dtype=jnp.float32)
            l0 = jnp.zeros((H, 1), dtype=jnp.float32)
            a0 = jnp.zeros((H, D), dtype=jnp.float32)
            m, l, acc = lax.fori_loop(0, n_steps, step_fn, (m0, l0, a0))

            comm_send[0, i] = acc
            comm_send[1, i] = jnp.broadcast_to(m, (H, D))
            comm_send[2, i] = jnp.broadcast_to(l, (H, D))

        rdma_x = pltpu.make_async_remote_copy(
            src_ref=comm_send, dst_ref=comm_recv,
            send_sem=send_sem, recv_sem=recv_sem,
            device_id=x_peer, device_id_type=pl.DeviceIdType.MESH,
        )
        rdma_x.start()
        rdma_x.wait()

        acc_a, m_a, l_a = comm_send[0], comm_send[1], comm_send[2]
        acc_b, m_b, l_b = comm_recv[0], comm_recv[1], comm_recv[2]
        m_star = jnp.maximum(m_a, m_b)
        aa = jnp.exp(m_a - m_star)
        ab = jnp.exp(m_b - m_star)
        l_star = l_a * aa + l_b * ab
        base = my_y * NB
        out_ref[pl.ds(base, NB)] = (acc_a * aa + acc_b * ab) / l_star

        rdma_y = pltpu.make_async_remote_copy(
            src_ref=out_ref.at[pl.ds(base, NB)],
            dst_ref=out_ref.at[pl.ds(base, NB)],
            send_sem=send_sem_y, recv_sem=recv_sem_y,
            device_id=y_peer, device_id_type=pl.DeviceIdType.MESH,
        )
        rdma_y.start()
        rdma_y.wait()

    out = pl.pallas_call(
        body,
        out_shape=jax.ShapeDtypeStruct((B, H, D), jnp.float32),
        in_specs=[
            pl.BlockSpec(memory_space=pltpu.VMEM),
            pl.BlockSpec(memory_space=pl.ANY),
            pl.BlockSpec(memory_space=pl.ANY),
            pl.BlockSpec(memory_space=pltpu.SMEM),
            pl.BlockSpec(memory_space=pltpu.SMEM),
        ],
        out_specs=pl.BlockSpec(memory_space=pltpu.VMEM),
        scratch_shapes=[
            pltpu.VMEM((2, G, BS, HD), jnp.float32),
            pltpu.VMEM((2, G, BS, HD), jnp.float32),
            pltpu.SemaphoreType.DMA((2,)),
            pltpu.SemaphoreType.DMA((2,)),
            pltpu.VMEM((3, NB, H, D), jnp.float32),
            pltpu.VMEM((3, NB, H, D), jnp.float32),
            pltpu.SemaphoreType.DMA,
            pltpu.SemaphoreType.DMA,
            pltpu.SemaphoreType.DMA,
            pltpu.SemaphoreType.DMA,
        ],
        compiler_params=pltpu.CompilerParams(collective_id=0),
    )(q_my, K, V, bt_my, counts_my)
    return out.reshape(B, 1, H, D)


# device time: 38365 ns/iter; 30.1794x vs baseline; 1.4221x over previous
import jax
import jax.numpy as jnp
from jax import lax
from jax.experimental import pallas as pl
from jax.experimental.pallas import tpu as pltpu

B, H, D, BS = 8, 8, 128, 16
HD = H * D
NP_LOCAL = 512
MAXLEN = 512
NB = B // 2
R = NB * H
CH = 32
NCH = NP_LOCAL // CH
CHK = CH * BS
NSLOT = 8
SCALE = D ** -0.5
NEG = -1e30


def kernel(Q, K, V, bt, lens):
    my_x = lax.axis_index("x")
    my_y = lax.axis_index("y")
    q = Q.reshape(B, H, D)

    row0 = my_y * NB
    q_my = lax.dynamic_slice(q, (row0, 0, 0), (NB, H, D))
    bt_my = lax.dynamic_slice(bt, (row0, 0), (NB, MAXLEN))
    lens_my = lax.dynamic_slice(lens, (row0,), (NB,))

    jidx = jnp.arange(MAXLEN, dtype=jnp.int32)[None, :]
    lidx = bt_my - my_x * NP_LOCAL
    owned = (jidx < lens_my[:, None]) & (lidx >= 0) & (lidx < NP_LOCAL)
    pag = jnp.where(owned, lidx, -1)
    piota = jnp.arange(NP_LOCAL, dtype=jnp.int32)

    eq = pag[:, :, None] == piota[None, None, :]
    ref_any = jnp.any(eq, axis=(0, 1))
    plist = jnp.sort(jnp.where(ref_any, piota, NP_LOCAL - 1))
    nd = jnp.sum(ref_any).astype(jnp.int32)

    w_pages = jnp.sum(
        (pag[:, :, None] == plist[None, None, :]), axis=1,
        dtype=jnp.float32)
    w_pages = jnp.where(jnp.arange(NP_LOCAL)[None, :] < nd, w_pages, 0.0)
    wp = w_pages.reshape(NB, NCH, CH).transpose(1, 0, 2)

    plist_s = plist.reshape(1, NP_LOCAL)
    nd_s = nd.reshape(1, 1)

    def body(q_ref, k_ref, v_ref, wp_ref, plist_ref, nd_ref, out_ref,
             kbuf, vbuf, ksems, vsems, comm_send, comm_recv,
             send_sem, recv_sem, send_sem_y, recv_sem_y):
        my_x = lax.axis_index("x")
        my_y = lax.axis_index("y")
        x_peer = (1 - my_x, my_y)
        y_peer = (my_x, 1 - my_y)

        barrier = pltpu.get_barrier_semaphore()
        for nbr in (x_peer, y_peer):
            pl.semaphore_signal(barrier, inc=1, device_id=nbr,
                                device_id_type=pl.DeviceIdType.MESH)
        pl.semaphore_wait(barrier, 2)

        kf_ref = k_ref.reshape(NP_LOCAL, BS, HD)
        vf_ref = v_ref.reshape(NP_LOCAL, BS, HD)

        n_chunks = (nd_ref[0, 0] + CH - 1) // CH

        def start_chunk(c, slot):
            for g in range(CH):
                pidx = plist_ref[0, c * CH + g]
                pltpu.make_async_copy(
                    kf_ref.at[pidx], kbuf.at[slot, g], ksems.at[slot]).start()
                pltpu.make_async_copy(
                    vf_ref.at[pidx], vbuf.at[slot, g], vsems.at[slot]).start()

        for d in range(NSLOT - 1):
            @pl.when(d < n_chunks)
            def _(d=d):
                start_chunk(d, d)

        rowh = lax.broadcasted_iota(jnp.int32, (R, HD), 0) % H
        colh = lax.broadcasted_iota(jnp.int32, (R, HD), 1) // D
        diag = rowh == colh
        q_cat = jnp.concatenate(
            [jnp.concatenate([q_ref[i]] * H, axis=1) for i in range(NB)],
            axis=0)
        q_bd = jnp.where(diag, q_cat, 0.0)

        e_mat = (lax.broadcasted_iota(jnp.int32, (CH, CHK), 1) // BS
                 == lax.broadcasted_iota(jnp.int32, (CH, CHK), 0)
                 ).astype(jnp.float32)
        b_mat = (lax.broadcasted_iota(jnp.int32, (R, NB), 0) // H
                 == lax.broadcasted_iota(jnp.int32, (R, NB), 1)
                 ).astype(jnp.float32)

        def step_fn(c, carry):
            m, l, acc = carry
            slot = lax.rem(c, NSLOT)

            @pl.when(c + NSLOT - 1 < n_chunks)
            def _():
                start_chunk(c + NSLOT - 1, lax.rem(c + NSLOT - 1, NSLOT))

            for _ in range(CH):
                pltpu.make_async_copy(
                    kf_ref.at[0], kbuf.at[slot, 0], ksems.at[slot]).wait()
                pltpu.make_async_copy(
                    vf_ref.at[0], vbuf.at[slot, 0], vsems.at[slot]).wait()

            k_all = kbuf[slot].reshape(CHK, HD)
            v_all = vbuf[slot].reshape(CHK, HD)
            s_mat = lax.dot_general(
                q_bd, k_all,
                (((1,), (1,)), ((), ())),
                preferred_element_type=jnp.float32,
            ) * SCALE

            wk = lax.dot_general(
                wp_ref[c], e_mat, (((1,), (0,)), ((), ())),
                preferred_element_type=jnp.float32)
            w_c = lax.dot_general(
                b_mat, wk, (((1,), (0,)), ((), ())),
                preferred_element_type=jnp.float32)
            s_eff = jnp.where(w_c > 0.0, s_mat, NEG)
            m_new = jnp.maximum(m, jnp.max(s_eff, axis=1, keepdims=True))
            alpha = jnp.exp(m - m_new)
            p = jnp.exp(s_eff - m_new) * w_c
            l_new = l * alpha + jnp.sum(p, axis=1, keepdims=True)
            r = lax.dot_general(
                p, v_all,
                (((1,), (0,)), ((), ())),
                preferred_element_type=jnp.float32,
            )
            rm = jnp.where(diag, r, 0.0)
            pv = rm[:, 0:D]
            for hh in range(1, H):
                pv = pv + rm[:, hh * D:(hh + 1) * D]
            acc_new = acc * alpha + pv
            return m_new, l_new, acc_new

        m0 = jnp.full((R, 1), NEG, dtype=jnp.float32)
        l0 = jnp.zeros((R, 1), dtype=jnp.float32)
        a0 = jnp.zeros((R, D), dtype=jnp.float32)
        m, l, acc = lax.fori_loop(0, n_chunks, step_fn, (m0, l0, a0))

        comm_send[0] = acc.reshape(NB, H, D)
        comm_send[1] = jnp.broadcast_to(m, (R, D)).reshape(NB, H, D)
        comm_send[2] = jnp.broadcast_to(l, (R, D)).reshape(NB, H, D)

        rdma_x = pltpu.make_async_remote_copy(
            src_ref=comm_send, dst_ref=comm_recv,
            send_sem=send_sem, recv_sem=recv_sem,
            device_id=x_peer, device_id_type=pl.DeviceIdType.MESH,
        )
        rdma_x.start()
        rdma_x.wait()

        acc_a, m_a, l_a = comm_send[0], comm_send[1], comm_send[2]
        acc_b, m_b, l_b = comm_recv[0], comm_recv[1], comm_recv[2]
        m_star = jnp.maximum(m_a, m_b)
        aa = jnp.exp(m_a - m_star)
        ab = jnp.exp(m_b - m_star)
        l_star = l_a * aa + l_b * ab
        base = my_y * NB
        out_ref[pl.ds(base, NB)] = (acc_a * aa + acc_b * ab) / l_star

        rdma_y = pltpu.make_async_remote_copy(
            src_ref=out_ref.at[pl.ds(base, NB)],
            dst_ref=out_ref.at[pl.ds(base, NB)],
            send_sem=send_sem_y, recv_sem=recv_sem_y,
            device_id=y_peer, device_id_type=pl.DeviceIdType.MESH,
        )
        rdma_y.start()
        rdma_y.wait()

    out = pl.pallas_call(
        body,
        out_shape=jax.ShapeDtypeStruct((B, H, D), jnp.float32),
        in_specs=[
            pl.BlockSpec(memory_space=pltpu.VMEM),
            pl.BlockSpec(memory_space=pl.ANY),
            pl.BlockSpec(memory_space=pl.ANY),
            pl.BlockSpec(memory_space=pltpu.VMEM),
            pl.BlockSpec(memory_space=pltpu.SMEM),
            pl.BlockSpec(memory_space=pltpu.SMEM),
        ],
        out_specs=pl.BlockSpec(memory_space=pltpu.VMEM),
        scratch_shapes=[
            pltpu.VMEM((NSLOT, CH, BS, HD), jnp.float32),
            pltpu.VMEM((NSLOT, CH, BS, HD), jnp.float32),
            pltpu.SemaphoreType.DMA((NSLOT,)),
            pltpu.SemaphoreType.DMA((NSLOT,)),
            pltpu.VMEM((3, NB, H, D), jnp.float32),
            pltpu.VMEM((3, NB, H, D), jnp.float32),
            pltpu.SemaphoreType.DMA,
            pltpu.SemaphoreType.DMA,
            pltpu.SemaphoreType.DMA,
            pltpu.SemaphoreType.DMA,
        ],
        compiler_params=pltpu.CompilerParams(
            collective_id=0, vmem_limit_bytes=60 * 1024 * 1024),
    )(q_my, K, V, wp, plist_s, nd_s)
    return out.reshape(B, 1, H, D)
